# Initial kernel scaffold; baseline (speedup 1.0000x reference)
import jax
import jax.numpy as jnp
from jax.experimental import pallas as pl


def kernel(features, segment_ids, rand_noise):
    raise NotImplementedError("write your pallas kernel here")


# speedup vs baseline: 2.0603x; 2.0603x over previous
import jax
import jax.numpy as jnp
from jax import lax
from jax.experimental import pallas as pl
from jax.experimental.pallas import tpu as pltpu
from jax.experimental.pallas import tpu_sc as plsc

N_ROWS = 32768
N_CH = 256
N_SEG = 16
N_CORES = 2
N_SUBCORES = 16
LANES = 16

CH_HALF = N_CH // N_CORES
ROWS_PER_TILE = N_ROWS // N_SUBCORES
CHUNK = 256
N_CHUNKS = ROWS_PER_TILE // CHUNK
SUB = 128
N_VECS = CH_HALF // LANES


def _rescale_body(feat_hbm, seg2d_hbm, noise_hbm, out_hbm,
                  featbuf, idx2d, pooled, noise_v, inv_v, acc):
    c = lax.axis_index("c")
    s = lax.axis_index("s")
    ch0 = c * CH_HALF
    row0 = s * ROWS_PER_TILE

    pltpu.sync_copy(
        seg2d_hbm.at[pl.ds(s * (ROWS_PER_TILE // SUB), ROWS_PER_TILE // SUB)],
        idx2d)

    @pl.when(s == 0)
    def _():
        def zbody(seg, carry):
            for v in range(N_VECS):
                pooled[seg, pl.ds(v * LANES, LANES)] = jnp.zeros(
                    (LANES,), jnp.float32)
            return carry
        lax.fori_loop(0, N_SEG, zbody, 0)
        pltpu.sync_copy(pooled, acc)
    plsc.subcore_barrier()

    def p1body(k, carry):
        pltpu.sync_copy(
            feat_hbm.at[pl.ds(row0 + k * CHUNK, CHUNK), pl.ds(ch0, CH_HALF)],
            featbuf)
        for j in range(CHUNK // SUB):
            pltpu.sync_copy(
                featbuf.at[pl.ds(j * SUB, SUB)],
                acc.at[idx2d.at[k * (CHUNK // SUB) + j]],
                add=True)
        return carry
    lax.fori_loop(0, N_CHUNKS, p1body, 0)
    plsc.subcore_barrier()

    pltpu.sync_copy(acc, pooled)
    pltpu.sync_copy(noise_hbm.at[pl.ds(0, N_SEG), pl.ds(ch0, CH_HALF)], noise_v)

    def p2body(seg, carry):
        for v in range(N_VECS):
            p = pooled[seg, pl.ds(v * LANES, LANES)]
            nz = noise_v[seg, pl.ds(v * LANES, LANES)]
            inv_v[seg, pl.ds(v * LANES, LANES)] = 1.0 / ((0.875 + 0.25 * nz) * p)
        return carry
    lax.fori_loop(0, N_SEG, p2body, 0)

    def p3body(k, carry):
        pltpu.sync_copy(
            feat_hbm.at[pl.ds(row0 + k * CHUNK, CHUNK), pl.ds(ch0, CH_HALF)],
            featbuf)

        def gbody(g, gcarry):
            gi = k * CHUNK + g * LANES
            segvec = idx2d[lax.shift_right_logical(gi, 7),
                           pl.ds(lax.bitwise_and(gi, SUB - 1), LANES)]
            base = g * LANES
            for i in range(LANES):
                seg = segvec[i]
                r = base + i
                for v in range(N_VECS):
                    featbuf[r, pl.ds(v * LANES, LANES)] = (
                        featbuf[r, pl.ds(v * LANES, LANES)]
                        * inv_v[seg, pl.ds(v * LANES, LANES)])
            return gcarry

        lax.fori_loop(0, CHUNK // LANES, gbody, 0)
        pltpu.sync_copy(
            featbuf,
            out_hbm.at[pl.ds(row0 + k * CHUNK, CHUNK), pl.ds(ch0, CH_HALF)])
        return carry
    lax.fori_loop(0, N_CHUNKS, p3body, 0)


def kernel(features, segment_ids, rand_noise):
    seg2d = segment_ids.astype(jnp.int32).reshape(N_ROWS // SUB, SUB)
    mesh = plsc.VectorSubcoreMesh(core_axis_name="c", subcore_axis_name="s")
    run = pl.kernel(
        _rescale_body,
        mesh=mesh,
        out_type=jax.ShapeDtypeStruct((N_ROWS, N_CH), jnp.float32),
        scratch_types=[
            pltpu.VMEM((CHUNK, CH_HALF), jnp.float32),
            pltpu.VMEM((ROWS_PER_TILE // SUB, SUB), jnp.int32),
            pltpu.VMEM((N_SEG, CH_HALF), jnp.float32),
            pltpu.VMEM((N_SEG, CH_HALF), jnp.float32),
            pltpu.VMEM((N_SEG, CH_HALF), jnp.float32),
            pltpu.VMEM_SHARED((N_SEG, CH_HALF), jnp.float32),
        ],
    )
    return run(features, seg2d, rand_noise)


# speedup vs baseline: 2.4389x; 1.1838x over previous
import jax
import jax.numpy as jnp
from jax import lax
from jax.experimental import pallas as pl
from jax.experimental.pallas import tpu as pltpu
from jax.experimental.pallas import tpu_sc as plsc

N_ROWS = 32768
N_CH = 256
N_SEG = 16
N_CORES = 2
N_SUBCORES = 16
LANES = 16

CH_HALF = N_CH // N_CORES
ROWS_PER_TILE = N_ROWS // N_SUBCORES
CHUNK = 128
N_CHUNKS = ROWS_PER_TILE // CHUNK
RING = 4
N_VECS = CH_HALF // LANES
GROUPS = CHUNK // LANES


def _rescale_body(feat_hbm, seg2d_hbm, noise_hbm, out_hbm,
                  ring, idx2d, pooled, noise_v, inv_v, acc,
                  in0, in1, in2, in3, ot0, ot1, ot2, ot3):
    in_sems = (in0, in1, in2, in3)
    out_sems = (ot0, ot1, ot2, ot3)
    c = lax.axis_index("c")
    s = lax.axis_index("s")
    ch0 = c * CH_HALF
    row0 = s * ROWS_PER_TILE

    def feat_src(k):
        return feat_hbm.at[pl.ds(row0 + k * CHUNK, CHUNK), pl.ds(ch0, CH_HALF)]

    def out_dst(k):
        return out_hbm.at[pl.ds(row0 + k * CHUNK, CHUNK), pl.ds(ch0, CH_HALF)]

    def slot(j):
        return ring.at[pl.ds(j * CHUNK, CHUNK)]

    def start_in(k, j):
        pltpu.async_copy(feat_src(k), slot(j), in_sems[j])

    def wait_in(j):
        pltpu.make_async_copy(feat_src(0), slot(j), in_sems[j]).wait()

    def start_out(k, j):
        pltpu.async_copy(slot(j), out_dst(k), out_sems[j])

    def wait_out(j):
        pltpu.make_async_copy(slot(j), out_dst(0), out_sems[j]).wait()

    pltpu.sync_copy(
        seg2d_hbm.at[pl.ds(s * (ROWS_PER_TILE // CHUNK), ROWS_PER_TILE // CHUNK)],
        idx2d)

    @pl.when(s == 0)
    def _():
        def zbody(seg, carry):
            for v in range(N_VECS):
                pooled[seg, pl.ds(v * LANES, LANES)] = jnp.zeros(
                    (LANES,), jnp.float32)
            return carry
        lax.fori_loop(0, N_SEG, zbody, 0)
        pltpu.sync_copy(pooled, acc)
    plsc.subcore_barrier()

    start_in(0, 0)
    start_in(1, 1)

    def p1body(q, carry):
        for j in range(RING):
            k = q * RING + j
            wait_in(j)
            nj = (j + 2) % RING
            if j < 2:
                start_in(k + 2, nj)
            else:
                @pl.when(q < (N_CHUNKS // RING) - 1)
                def _():
                    start_in(k + 2, nj)
            pltpu.sync_copy(slot(j), acc.at[idx2d.at[k]], add=True)
        return carry
    lax.fori_loop(0, N_CHUNKS // RING, p1body, 0)
    plsc.subcore_barrier()

    pltpu.sync_copy(acc, pooled)
    pltpu.sync_copy(noise_hbm.at[pl.ds(0, N_SEG), pl.ds(ch0, CH_HALF)], noise_v)

    def p2body(seg, carry):
        for v in range(N_VECS):
            p = pooled[seg, pl.ds(v * LANES, LANES)]
            nz = noise_v[seg, pl.ds(v * LANES, LANES)]
            inv_v[seg, pl.ds(v * LANES, LANES)] = 1.0 / ((0.875 + 0.25 * nz) * p)
        return carry
    lax.fori_loop(0, N_SEG, p2body, 0)

    def compute_chunk(k, j):
        base_j = j * CHUNK

        def gbody(g, gcarry):
            gi = k * CHUNK + g * LANES
            segvec = idx2d[lax.shift_right_logical(gi, 7),
                           pl.ds(lax.bitwise_and(gi, 127), LANES)]
            base = base_j + g * LANES
            for i in range(LANES):
                seg = segvec[i]
                for v in range(N_VECS):
                    col = v * LANES
                    ring[base + i, pl.ds(col, LANES)] = (
                        ring[base + i, pl.ds(col, LANES)]
                        * inv_v[seg, pl.ds(col, LANES)])
            return gcarry
        lax.fori_loop(0, GROUPS, gbody, 0)

    start_in(0, 0)
    start_in(1, 1)

    def p3body(q, carry):
        for j in range(RING):
            k = q * RING + j
            wait_in(j)
            compute_chunk(k, j)
            start_out(k, j)
            nj = (j + 2) % RING
            if j < 2:
                @pl.when(q > 0)
                def _():
                    wait_out(nj)
                start_in(k + 2, nj)
            else:
                @pl.when(q < (N_CHUNKS // RING) - 1)
                def _():
                    wait_out(nj)
                    start_in(k + 2, nj)
        return carry
    lax.fori_loop(0, N_CHUNKS // RING, p3body, 0)
    wait_out(0)
    wait_out(1)
    wait_out(2)
    wait_out(3)


def kernel(features, segment_ids, rand_noise):
    seg2d = segment_ids.astype(jnp.int32).reshape(N_ROWS // CHUNK, CHUNK)
    mesh = plsc.VectorSubcoreMesh(core_axis_name="c", subcore_axis_name="s")
    run = pl.kernel(
        _rescale_body,
        mesh=mesh,
        out_type=jax.ShapeDtypeStruct((N_ROWS, N_CH), jnp.float32),
        scratch_types=[
            pltpu.VMEM((RING * CHUNK, CH_HALF), jnp.float32),
            pltpu.VMEM((ROWS_PER_TILE // CHUNK, CHUNK), jnp.int32),
            pltpu.VMEM((N_SEG, CH_HALF), jnp.float32),
            pltpu.VMEM((N_SEG, CH_HALF), jnp.float32),
            pltpu.VMEM((N_SEG, CH_HALF), jnp.float32),
            pltpu.VMEM_SHARED((N_SEG, CH_HALF), jnp.float32),
        ] + [pltpu.SemaphoreType.DMA] * 8,
    )
    return run(features, seg2d, rand_noise)
